# Initial kernel scaffold; baseline (speedup 1.0000x reference)
import jax
import jax.numpy as jnp
from jax.experimental import pallas as pl


def kernel(x_lc, batch_lc, W1, b1, W2, b2, We1, be1, We2, be2, We3, be3, We4, be4, Wo1, bo1, Wo2, bo2):
    raise NotImplementedError("write your pallas kernel here")


# speedup vs baseline: 9.1335x; 9.1335x over previous
import jax
import jax.numpy as jnp
from jax import lax
from jax.experimental import pallas as pl

_M = 512
_H = 64
_K = 24


def _elu(v):
    vn = jnp.where(v > 0, 0.0, v)
    return jnp.where(v > 0, v, jnp.tanh(vn * 0.5) * (jnp.exp(vn) + 1.0))


def _mm(a, b):
    return lax.dot_general(a, b, (((1,), (0,)), ((), ())),
                           preferred_element_type=jnp.float32)


def _mm_seq(a, w):
    acc = a[:, 0:1] * w[0:1, :]
    for c in range(1, w.shape[0]):
        acc = acc + a[:, c:c + 1] * w[c:c + 1, :]
    return acc


def _edge_layer(x, we, be):
    xt = jnp.transpose(x)
    sq_t = jnp.sum(xt * xt, axis=0, keepdims=True)
    sq_r = jnp.transpose(sq_t)
    xx = lax.dot_general(x, x, (((1,), (1,)), ((), ())),
                         preferred_element_type=jnp.float32)
    d = (sq_r + sq_t) - 2.0 * xx

    h1 = x.astype(jnp.bfloat16).astype(jnp.float32)
    r1 = x - h1
    h2 = r1.astype(jnp.bfloat16).astype(jnp.float32)
    h3 = (r1 - h2).astype(jnp.bfloat16).astype(jnp.float32)
    g = jnp.concatenate([h1, h2, h3], axis=1)

    iota = lax.broadcasted_iota(jnp.int32, (_M, _M), 1)
    rev = _M - iota

    def step(_, carry):
        dc, om = carry
        m = jnp.min(dc, axis=1, keepdims=True)
        r = jnp.max(jnp.where(dc == m, rev, 0), axis=1, keepdims=True)
        oh = rev == r
        dc = jnp.where(oh, jnp.float32(1e30), dc)
        p = _mm(oh.astype(jnp.float32), g)
        xj = (p[:, :_H] + p[:, _H:2 * _H]) + p[:, 2 * _H:]
        cat = jnp.concatenate([x, xj - x], axis=1)
        om = jnp.maximum(om, _elu(_mm(cat, we) + be))
        return dc, om

    _, om = lax.fori_loop(
        0, _K, step, (d, jnp.full((_M, _H), -1e30, jnp.float32)))
    return om


def _net_body(x_ref, w1, b1, w2, b2,
              we1, be1, we2, be2, we3, be3, we4, be4,
              wo1, bo1, wo2, bo2, out_ref):
    x = _elu(_mm(x_ref[...], w1[...]) + b1[...])
    x = _elu(_mm(x, w2[...]) + b2[...])
    x = _edge_layer(x, we1[...], be1[...])
    x = _edge_layer(x, we2[...], be2[...])
    x = _edge_layer(x, we3[...], be3[...])
    x = _edge_layer(x, we4[...], be4[...])
    y = _elu(_mm(x, wo1[...]) + bo1[...])
    out_ref[...] = _mm(y, wo2[...]) + bo2[...]


def kernel(x_lc, batch_lc, W1, b1, W2, b2, We1, be1, We2, be2,
           We3, be3, We4, be4, Wo1, bo1, Wo2, bo2):
    n = x_lc.shape[0]
    g = n // _M

    def row(v):
        return v.reshape(1, -1)

    full = lambda s: pl.BlockSpec(s, lambda i: (0, 0))
    in_specs = [pl.BlockSpec((_M, x_lc.shape[1]), lambda i: (i, 0))]
    args = [x_lc]
    for w in (W1, row(b1), W2, row(b2),
              We1, row(be1), We2, row(be2),
              We3, row(be3), We4, row(be4),
              Wo1, row(bo1), Wo2, row(bo2)):
        in_specs.append(full(w.shape))
        args.append(w)

    out = pl.pallas_call(
        _net_body,
        grid=(g,),
        in_specs=in_specs,
        out_specs=pl.BlockSpec((_M, Wo2.shape[1]), lambda i: (i, 0)),
        out_shape=jax.ShapeDtypeStruct((n, Wo2.shape[1]), jnp.float32),
    )(*args)
    return (out, batch_lc)


# speedup vs baseline: 10.2211x; 1.1191x over previous
import jax
import jax.numpy as jnp
from jax import lax
from jax.experimental import pallas as pl

_M = 512
_H = 64
_K = 24


def _elu(v):
    vn = jnp.where(v > 0, 0.0, v)
    return jnp.where(v > 0, v, jnp.tanh(vn * 0.5) * (jnp.exp(vn) + 1.0))


def _mm(a, b):
    return lax.dot_general(a, b, (((1,), (0,)), ((), ())),
                           preferred_element_type=jnp.float32)


def _mm_seq(a, w):
    acc = a[:, 0:1] * w[0:1, :]
    for c in range(1, w.shape[0]):
        acc = acc + a[:, c:c + 1] * w[c:c + 1, :]
    return acc


def _edge_layer(x, we, be):
    xt = jnp.transpose(x)
    sq_t = jnp.sum(xt * xt, axis=0, keepdims=True)
    sq_r = jnp.transpose(sq_t)
    xx = lax.dot_general(x, x, (((1,), (1,)), ((), ())),
                         preferred_element_type=jnp.float32)
    d = (sq_r + sq_t) - 2.0 * xx

    h1 = x.astype(jnp.bfloat16).astype(jnp.float32)
    r1 = x - h1
    h2 = r1.astype(jnp.bfloat16).astype(jnp.float32)
    h3 = (r1 - h2).astype(jnp.bfloat16).astype(jnp.float32)
    g = jnp.concatenate([h1, h2, h3], axis=1)

    iota = lax.broadcasted_iota(jnp.int32, (_M, _M), 1)
    rev = _M - iota

    def extract(dc):
        m = jnp.min(dc, axis=1, keepdims=True)
        r = jnp.max(jnp.where(dc == m, rev, 0), axis=1, keepdims=True)
        oh = rev == r
        return r, jnp.where(oh, jnp.float32(1e30), dc)

    def aggregate(om, r):
        oh = rev == r
        p = _mm(oh.astype(jnp.float32), g)
        xj = (p[:, :_H] + p[:, _H:2 * _H]) + p[:, 2 * _H:]
        cat = jnp.concatenate([x, xj - x], axis=1)
        return jnp.maximum(om, _elu(_mm(cat, we) + be))

    r0, d1 = extract(d)

    def step(_, carry):
        dc, om, rp = carry
        r, dc2 = extract(dc)
        om2 = aggregate(om, rp)
        return dc2, om2, r

    _, om, rl = lax.fori_loop(
        0, _K - 1, step,
        (d1, jnp.full((_M, _H), -1e30, jnp.float32), r0))
    return aggregate(om, rl)


def _net_body(x_ref, w1, b1, w2, b2,
              we1, be1, we2, be2, we3, be3, we4, be4,
              wo1, bo1, wo2, bo2, out_ref):
    x = _elu(_mm(x_ref[...], w1[...]) + b1[...])
    x = _elu(_mm(x, w2[...]) + b2[...])
    x = _edge_layer(x, we1[...], be1[...])
    x = _edge_layer(x, we2[...], be2[...])
    x = _edge_layer(x, we3[...], be3[...])
    x = _edge_layer(x, we4[...], be4[...])
    y = _elu(_mm(x, wo1[...]) + bo1[...])
    out_ref[...] = _mm(y, wo2[...]) + bo2[...]


def kernel(x_lc, batch_lc, W1, b1, W2, b2, We1, be1, We2, be2,
           We3, be3, We4, be4, Wo1, bo1, Wo2, bo2):
    n = x_lc.shape[0]
    g = n // _M

    def row(v):
        return v.reshape(1, -1)

    full = lambda s: pl.BlockSpec(s, lambda i: (0, 0))
    in_specs = [pl.BlockSpec((_M, x_lc.shape[1]), lambda i: (i, 0))]
    args = [x_lc]
    for w in (W1, row(b1), W2, row(b2),
              We1, row(be1), We2, row(be2),
              We3, row(be3), We4, row(be4),
              Wo1, row(bo1), Wo2, row(bo2)):
        in_specs.append(full(w.shape))
        args.append(w)

    out = pl.pallas_call(
        _net_body,
        grid=(g,),
        in_specs=in_specs,
        out_specs=pl.BlockSpec((_M, Wo2.shape[1]), lambda i: (i, 0)),
        out_shape=jax.ShapeDtypeStruct((n, Wo2.shape[1]), jnp.float32),
    )(*args)
    return (out, batch_lc)
